# baseline (speedup 1.0000x reference)
import jax
import jax.numpy as jnp
from jax import lax
from jax.experimental import pallas as pl
from jax.experimental.pallas import tpu as pltpu
from jax.experimental.pallas import tpu_sc as plsc
from jax.experimental.layout import Layout, Format

N_SAMPLES = 2_000_000
NCLS = 8
NCORES = 2
NSUB = 16
NW = NCORES * NSUB
UNROLL = 4
SUPER = 61
CHUNK = SUPER * 16 * UNROLL
NCHUNKS = 16
EPIA = 32
PER_TILE = CHUNK * NCHUNKS + EPIA
TAIL = N_SAMPLES - PER_TILE * NW
TAIL_BASE = PER_TILE * NW
NBINS = NCLS * NCLS


def _body(preds_hbm, true_hbm, out_hbm, pbuf, lbuf, h0, h1, h2, h3, outv):
    cid = lax.axis_index("c")
    sid = lax.axis_index("s")
    wid = sid * NCORES + cid
    hists = [h0, h1, h2, h3]

    lanes = lax.iota(jnp.int32, 16)
    ones = jnp.full((16,), 1.0, jnp.float32)
    cvecs = [lanes * NCLS + j for j in range(NCLS)]
    lane_rows = lanes * NBINS

    zero = jnp.zeros((16,), jnp.float32)
    for h in hists:
        for k in range(NBINS):
            h[pl.ds(k * 16, 16)] = zero

    def step16(histref, fb, lt):
        fbn = fb * NCLS
        cols = [plsc.load_gather(pbuf, [fbn + cvecs[j]]) for j in range(NCLS)]
        es = [jnp.exp(c) for c in cols]
        s = es[0]
        for j in range(1, NCLS):
            s = s + es[j]
        w = es[1]
        for j in range(2, NCLS):
            w = w + jnp.float32(j) * es[j]
        pred = (w / s + jnp.float32(0.5)).astype(jnp.int32)
        pred = jnp.minimum(jnp.maximum(pred, 0), NCLS - 1)
        lab = lbuf[pl.ds(lt, 16)]
        b = lab * NCLS + pred
        plsc.addupdate_scatter(histref, [lane_rows + b], ones)

    def sblock(t, _):
        base = t * (UNROLL * 16)
        for u in range(UNROLL):
            step16(hists[u], base + u * 16, base + u * 16)
        return 0

    def chunk_loop(c, _):
        base_row = wid * PER_TILE + c * CHUNK
        pltpu.sync_copy(
            preds_hbm.at[pl.ds(base_row * NCLS, CHUNK * NCLS)], pbuf
        )
        pltpu.sync_copy(true_hbm.at[pl.ds(base_row, CHUNK)], lbuf)
        lax.fori_loop(0, SUPER, sblock, 0)
        return 0

    lax.fori_loop(0, NCHUNKS, chunk_loop, 0)

    epi_row = wid * PER_TILE + CHUNK * NCHUNKS
    pltpu.sync_copy(
        preds_hbm.at[pl.ds(epi_row * NCLS, EPIA * NCLS)],
        pbuf.at[pl.ds(0, EPIA * NCLS)],
    )
    pltpu.sync_copy(true_hbm.at[pl.ds(epi_row, EPIA)], lbuf.at[pl.ds(0, EPIA)])
    for u in range(EPIA // 16):
        step16(hists[u], u * 16, u * 16)

    @pl.when(wid == 0)
    def _():
        pltpu.sync_copy(
            preds_hbm.at[pl.ds(TAIL_BASE * NCLS, TAIL * NCLS)],
            pbuf.at[pl.ds(0, TAIL * NCLS)],
        )
        pltpu.sync_copy(
            true_hbm.at[pl.ds(TAIL_BASE, TAIL)], lbuf.at[pl.ds(0, TAIL)]
        )
        for u in range(TAIL // 16):
            step16(hists[u % UNROLL], u * 16, u * 16)

    for g in range(NBINS // 16):
        tot = zero
        for h in hists:
            for l in range(16):
                tot = tot + h[pl.ds(l * NBINS + g * 16, 16)]
        outv[pl.ds(g * 16, 16)] = tot
    pltpu.sync_copy(outv, out_hbm.at[pl.ds(wid * NBINS, NBINS)])


@jax.jit
def _sc_counts(preds_flat, true_i32):
    mesh = plsc.VectorSubcoreMesh(core_axis_name="c", subcore_axis_name="s")
    fn = pl.kernel(
        _body,
        out_type=jax.ShapeDtypeStruct((NW * NBINS,), jnp.float32),
        mesh=mesh,
        scratch_types=[
            pltpu.VMEM((CHUNK * NCLS,), jnp.float32),
            pltpu.VMEM((CHUNK,), jnp.int32),
            pltpu.VMEM((16 * NBINS,), jnp.float32),
            pltpu.VMEM((16 * NBINS,), jnp.float32),
            pltpu.VMEM((16 * NBINS,), jnp.float32),
            pltpu.VMEM((16 * NBINS,), jnp.float32),
            pltpu.VMEM((NBINS,), jnp.float32),
        ],
        compiler_params=pltpu.CompilerParams(needs_layout_passes=False),
    )
    return fn(preds_flat, true_i32)


def kernel(preds, true):
    true_i32 = true.astype(jnp.int32)
    lin = Format(
        Layout(major_to_minor=(1, 0), tiling=()),
        jax.sharding.SingleDeviceSharding(jax.devices()[0]),
    )
    preds_lin = jax.device_put(preds, lin)
    flat = preds_lin.reshape(-1)
    rows = _sc_counts(flat, true_i32)
    counts = rows.reshape(NW, NBINS).sum(axis=0).reshape(NCLS, NCLS)
    i = jnp.arange(NCLS, dtype=jnp.float32)
    weights = (i[:, None] - i[None, :]) ** 2 / float((NCLS - 1) ** 2)
    total = counts.sum()
    th = counts.sum(axis=1)
    ph = counts.sum(axis=0)
    num = (counts * weights).sum() / total
    e = jnp.outer(th, ph)
    den = (e * weights).sum() / e.sum()
    return num / den


# speedup vs baseline: 1.2830x; 1.2830x over previous
import functools

import jax
import jax.numpy as jnp
from jax import lax
from jax.experimental import pallas as pl
from jax.experimental.pallas import tpu as pltpu
from jax.experimental.pallas import tpu_sc as plsc

N_SAMPLES = 2_000_000
NCLS = 8
NBINS = NCLS * NCLS


TC_BLOCK = 32768
TC_GRID = -(-N_SAMPLES // TC_BLOCK)


def _tc_body(p_ref, t_ref, o_ref):
    x = p_ref[...]
    xt = x.T
    e = jnp.exp(xt)
    wt = jnp.concatenate(
        [
            jnp.ones((1, NCLS), jnp.float32),
            lax.broadcasted_iota(jnp.int32, (1, NCLS), 1).astype(jnp.float32),
        ],
        axis=0,
    )
    r = lax.dot_general(
        wt, e, (((1,), (0,)), ((), ())),
        preferred_element_type=jnp.float32,
    )
    q = r[1:2, :] / r[0:1, :] + jnp.float32(0.5)
    pred = jnp.clip(q.astype(jnp.int32), 0, NCLS - 1)
    bins = t_ref[...].reshape(1, TC_BLOCK) * NCLS + pred
    o_ref[...] = bins.reshape(TC_BLOCK)


@jax.jit
def _tc_bins(preds, true_i32):
    return pl.pallas_call(
        _tc_body,
        grid=(TC_GRID,),
        in_specs=[
            pl.BlockSpec((TC_BLOCK, NCLS), lambda i: (i, 0)),
            pl.BlockSpec((TC_BLOCK,), lambda i: (i,)),
        ],
        out_specs=pl.BlockSpec((TC_BLOCK,), lambda i: (i,)),
        out_shape=jax.ShapeDtypeStruct((N_SAMPLES,), jnp.int32),
    )(preds, true_i32)


NCORES = 2
NSUB = 16
NW = NCORES * NSUB
UNROLL = 4
SUPER = 61
CHUNK = SUPER * 16 * UNROLL
NCHUNKS = 16
EPIA = 32
PER_TILE = CHUNK * NCHUNKS + EPIA
TAIL = N_SAMPLES - PER_TILE * NW
TAIL_BASE = PER_TILE * NW


def _sc_body(bins_hbm, out_hbm, lbuf, h0, h1, h2, h3, outv):
    cid = lax.axis_index("c")
    sid = lax.axis_index("s")
    wid = sid * NCORES + cid
    hists = [h0, h1, h2, h3]

    lanes = lax.iota(jnp.int32, 16)
    ones = jnp.full((16,), 1.0, jnp.float32)
    lane_rows = lanes * NBINS

    zero = jnp.zeros((16,), jnp.float32)
    for h in hists:
        for k in range(NBINS):
            h[pl.ds(k * 16, 16)] = zero

    def step16(histref, lt):
        b = lbuf[pl.ds(lt, 16)]
        plsc.addupdate_scatter(histref, [lane_rows + b], ones)

    def sblock(t, _):
        base = t * (UNROLL * 16)
        for u in range(UNROLL):
            step16(hists[u], base + u * 16)
        return 0

    def chunk_loop(c, _):
        base = wid * PER_TILE + c * CHUNK
        pltpu.sync_copy(bins_hbm.at[pl.ds(base, CHUNK)], lbuf)
        lax.fori_loop(0, SUPER, sblock, 0)
        return 0

    lax.fori_loop(0, NCHUNKS, chunk_loop, 0)

    epi = wid * PER_TILE + CHUNK * NCHUNKS
    pltpu.sync_copy(bins_hbm.at[pl.ds(epi, EPIA)], lbuf.at[pl.ds(0, EPIA)])
    for u in range(EPIA // 16):
        step16(hists[u], u * 16)

    @pl.when(wid == 0)
    def _():
        pltpu.sync_copy(
            bins_hbm.at[pl.ds(TAIL_BASE, TAIL)], lbuf.at[pl.ds(0, TAIL)]
        )
        for u in range(TAIL // 16):
            step16(hists[u % UNROLL], u * 16)

    for g in range(NBINS // 16):
        tot = zero
        for h in hists:
            for l in range(16):
                tot = tot + h[pl.ds(l * NBINS + g * 16, 16)]
        outv[pl.ds(g * 16, 16)] = tot
    pltpu.sync_copy(outv, out_hbm.at[pl.ds(wid * NBINS, NBINS)])


@jax.jit
def _sc_counts(bins):
    mesh = plsc.VectorSubcoreMesh(core_axis_name="c", subcore_axis_name="s")
    fn = pl.kernel(
        _sc_body,
        out_type=jax.ShapeDtypeStruct((NW * NBINS,), jnp.float32),
        mesh=mesh,
        scratch_types=[
            pltpu.VMEM((CHUNK,), jnp.int32),
            pltpu.VMEM((16 * NBINS,), jnp.float32),
            pltpu.VMEM((16 * NBINS,), jnp.float32),
            pltpu.VMEM((16 * NBINS,), jnp.float32),
            pltpu.VMEM((16 * NBINS,), jnp.float32),
            pltpu.VMEM((NBINS,), jnp.float32),
        ],
        compiler_params=pltpu.CompilerParams(needs_layout_passes=False),
    )
    return fn(bins)


def kernel(preds, true):
    true_i32 = true.astype(jnp.int32)
    bins = _tc_bins(preds, true_i32)
    rows = _sc_counts(bins)
    counts = rows.reshape(NW, NBINS).sum(axis=0).reshape(NCLS, NCLS)
    i = jnp.arange(NCLS, dtype=jnp.float32)
    weights = (i[:, None] - i[None, :]) ** 2 / float((NCLS - 1) ** 2)
    total = counts.sum()
    th = counts.sum(axis=1)
    ph = counts.sum(axis=0)
    num = (counts * weights).sum() / total
    e = jnp.outer(th, ph)
    den = (e * weights).sum() / e.sum()
    return num / den


# speedup vs baseline: 1.2994x; 1.0128x over previous
import functools

import jax
import jax.numpy as jnp
from jax import lax
from jax.experimental import pallas as pl
from jax.experimental.pallas import tpu as pltpu
from jax.experimental.pallas import tpu_sc as plsc

N_SAMPLES = 2_000_000
NCLS = 8
NBINS = NCLS * NCLS


TC_BLOCK = 49152
TC_GRID = -(-N_SAMPLES // TC_BLOCK)


def _tc_body(p_ref, t_ref, o_ref):
    x = p_ref[...]
    xt = x.T
    e = jnp.exp(xt)
    wt = jnp.concatenate(
        [
            jnp.ones((1, NCLS), jnp.float32),
            lax.broadcasted_iota(jnp.int32, (1, NCLS), 1).astype(jnp.float32),
        ],
        axis=0,
    )
    r = lax.dot_general(
        wt, e, (((1,), (0,)), ((), ())),
        preferred_element_type=jnp.float32,
    )
    q = r[1:2, :] / r[0:1, :] + jnp.float32(0.5)
    pred = jnp.clip(q.astype(jnp.int32), 0, NCLS - 1)
    bins = t_ref[...].reshape(1, TC_BLOCK) * NCLS + pred
    o_ref[...] = bins.reshape(TC_BLOCK)


@jax.jit
def _tc_bins(preds, true_i32):
    return pl.pallas_call(
        _tc_body,
        grid=(TC_GRID,),
        in_specs=[
            pl.BlockSpec((TC_BLOCK, NCLS), lambda i: (i, 0)),
            pl.BlockSpec((TC_BLOCK,), lambda i: (i,)),
        ],
        out_specs=pl.BlockSpec((TC_BLOCK,), lambda i: (i,)),
        out_shape=jax.ShapeDtypeStruct((N_SAMPLES,), jnp.int32),
    )(preds, true_i32)


NCORES = 2
NSUB = 16
NW = NCORES * NSUB
UNROLL = 4
SUPER = 61
CHUNK = SUPER * 16 * UNROLL
NCHUNKS = 16
EPIA = 32
PER_TILE = CHUNK * NCHUNKS + EPIA
TAIL = N_SAMPLES - PER_TILE * NW
TAIL_BASE = PER_TILE * NW


def _sc_body(bins_hbm, out_hbm, lbuf, h0, h1, h2, h3, outv):
    cid = lax.axis_index("c")
    sid = lax.axis_index("s")
    wid = sid * NCORES + cid
    hists = [h0, h1, h2, h3]

    lanes = lax.iota(jnp.int32, 16)
    ones = jnp.full((16,), 1.0, jnp.float32)
    lane_rows = lanes * NBINS

    zero = jnp.zeros((16,), jnp.float32)
    for h in hists:
        for k in range(NBINS):
            h[pl.ds(k * 16, 16)] = zero

    def step16(histref, lt):
        b = lbuf[pl.ds(lt, 16)]
        plsc.addupdate_scatter(histref, [lane_rows + b], ones)

    def sblock(t, _):
        base = t * (UNROLL * 16)
        for u in range(UNROLL):
            step16(hists[u], base + u * 16)
        return 0

    def chunk_loop(c, _):
        base = wid * PER_TILE + c * CHUNK
        pltpu.sync_copy(bins_hbm.at[pl.ds(base, CHUNK)], lbuf)
        lax.fori_loop(0, SUPER, sblock, 0)
        return 0

    lax.fori_loop(0, NCHUNKS, chunk_loop, 0)

    epi = wid * PER_TILE + CHUNK * NCHUNKS
    pltpu.sync_copy(bins_hbm.at[pl.ds(epi, EPIA)], lbuf.at[pl.ds(0, EPIA)])
    for u in range(EPIA // 16):
        step16(hists[u], u * 16)

    @pl.when(wid == 0)
    def _():
        pltpu.sync_copy(
            bins_hbm.at[pl.ds(TAIL_BASE, TAIL)], lbuf.at[pl.ds(0, TAIL)]
        )
        for u in range(TAIL // 16):
            step16(hists[u % UNROLL], u * 16)

    for g in range(NBINS // 16):
        tot = zero
        for h in hists:
            for l in range(16):
                tot = tot + h[pl.ds(l * NBINS + g * 16, 16)]
        outv[pl.ds(g * 16, 16)] = tot
    pltpu.sync_copy(outv, out_hbm.at[pl.ds(wid * NBINS, NBINS)])


@jax.jit
def _sc_counts(bins):
    mesh = plsc.VectorSubcoreMesh(core_axis_name="c", subcore_axis_name="s")
    fn = pl.kernel(
        _sc_body,
        out_type=jax.ShapeDtypeStruct((NW * NBINS,), jnp.float32),
        mesh=mesh,
        scratch_types=[
            pltpu.VMEM((CHUNK,), jnp.int32),
            pltpu.VMEM((16 * NBINS,), jnp.float32),
            pltpu.VMEM((16 * NBINS,), jnp.float32),
            pltpu.VMEM((16 * NBINS,), jnp.float32),
            pltpu.VMEM((16 * NBINS,), jnp.float32),
            pltpu.VMEM((NBINS,), jnp.float32),
        ],
        compiler_params=pltpu.CompilerParams(needs_layout_passes=False),
    )
    return fn(bins)


def kernel(preds, true):
    true_i32 = true.astype(jnp.int32)
    bins = _tc_bins(preds, true_i32)
    rows = _sc_counts(bins)
    counts = rows.reshape(NW, NBINS).sum(axis=0).reshape(NCLS, NCLS)
    i = jnp.arange(NCLS, dtype=jnp.float32)
    weights = (i[:, None] - i[None, :]) ** 2 / float((NCLS - 1) ** 2)
    total = counts.sum()
    th = counts.sum(axis=1)
    ph = counts.sum(axis=0)
    num = (counts * weights).sum() / total
    e = jnp.outer(th, ph)
    den = (e * weights).sum() / e.sum()
    return num / den


# speedup vs baseline: 1.3029x; 1.0027x over previous
import jax
import jax.numpy as jnp
from jax import lax
from jax.experimental import pallas as pl
from jax.experimental.pallas import tpu as pltpu
from jax.experimental.pallas import tpu_sc as plsc

N_SAMPLES = 2_000_000
NCLS = 8
NBINS = NCLS * NCLS


TC_BLOCK = 57344
TC_GRID = -(-N_SAMPLES // TC_BLOCK)


def _tc_body(p_ref, t_ref, o_ref):
    x = p_ref[...]
    xt = x.T
    e = jnp.exp(xt)
    wt = jnp.concatenate(
        [
            jnp.ones((1, NCLS), jnp.float32),
            lax.broadcasted_iota(jnp.int32, (1, NCLS), 1).astype(jnp.float32),
        ],
        axis=0,
    )
    r = lax.dot_general(
        wt, e, (((1,), (0,)), ((), ())),
        preferred_element_type=jnp.float32,
    )
    q = r[1:2, :] / r[0:1, :] + jnp.float32(0.5)
    pred = jnp.clip(q.astype(jnp.int32), 0, NCLS - 1)
    bins = t_ref[...].reshape(1, TC_BLOCK) * NCLS + pred
    o_ref[...] = bins.reshape(TC_BLOCK)


@jax.jit
def _tc_bins(preds, true_i32):
    return pl.pallas_call(
        _tc_body,
        grid=(TC_GRID,),
        in_specs=[
            pl.BlockSpec((TC_BLOCK, NCLS), lambda i: (i, 0)),
            pl.BlockSpec((TC_BLOCK,), lambda i: (i,)),
        ],
        out_specs=pl.BlockSpec((TC_BLOCK,), lambda i: (i,)),
        out_shape=jax.ShapeDtypeStruct((N_SAMPLES,), jnp.int32),
    )(preds, true_i32)


NCORES = 2
NSUB = 16
NW = NCORES * NSUB
UNROLL = 4
SUPER = 61
CHUNK = SUPER * 16 * UNROLL
NCHUNKS = 16
EPIA = 32
PER_TILE = CHUNK * NCHUNKS + EPIA
TAIL = N_SAMPLES - PER_TILE * NW
TAIL_BASE = PER_TILE * NW


def _sc_body(bins_hbm, out_hbm, lbuf, h0, h1, h2, h3, outv):
    cid = lax.axis_index("c")
    sid = lax.axis_index("s")
    wid = sid * NCORES + cid
    hists = [h0, h1, h2, h3]

    lanes = lax.iota(jnp.int32, 16)
    ones = jnp.full((16,), 1.0, jnp.float32)
    lane_rows = lanes * NBINS

    zero = jnp.zeros((16,), jnp.float32)
    for h in hists:
        for k in range(NBINS):
            h[pl.ds(k * 16, 16)] = zero

    def step16(histref, lt):
        b = lbuf[pl.ds(lt, 16)]
        plsc.addupdate_scatter(histref, [lane_rows + b], ones)

    def sblock(t, _):
        base = t * (UNROLL * 16)
        for u in range(UNROLL):
            step16(hists[u], base + u * 16)
        return 0

    def chunk_loop(c, _):
        base = wid * PER_TILE + c * CHUNK
        pltpu.sync_copy(bins_hbm.at[pl.ds(base, CHUNK)], lbuf)
        lax.fori_loop(0, SUPER, sblock, 0)
        return 0

    lax.fori_loop(0, NCHUNKS, chunk_loop, 0)

    epi = wid * PER_TILE + CHUNK * NCHUNKS
    pltpu.sync_copy(bins_hbm.at[pl.ds(epi, EPIA)], lbuf.at[pl.ds(0, EPIA)])
    for u in range(EPIA // 16):
        step16(hists[u], u * 16)

    @pl.when(wid == 0)
    def _():
        pltpu.sync_copy(
            bins_hbm.at[pl.ds(TAIL_BASE, TAIL)], lbuf.at[pl.ds(0, TAIL)]
        )
        for u in range(TAIL // 16):
            step16(hists[u % UNROLL], u * 16)

    for g in range(NBINS // 16):
        tot = zero
        for h in hists:
            for l in range(16):
                tot = tot + h[pl.ds(l * NBINS + g * 16, 16)]
        outv[pl.ds(g * 16, 16)] = tot
    pltpu.sync_copy(outv, out_hbm.at[pl.ds(wid * NBINS, NBINS)])


@jax.jit
def _sc_counts(bins):
    mesh = plsc.VectorSubcoreMesh(core_axis_name="c", subcore_axis_name="s")
    fn = pl.kernel(
        _sc_body,
        out_type=jax.ShapeDtypeStruct((NW * NBINS,), jnp.float32),
        mesh=mesh,
        scratch_types=[
            pltpu.VMEM((CHUNK,), jnp.int32),
            pltpu.VMEM((16 * NBINS,), jnp.float32),
            pltpu.VMEM((16 * NBINS,), jnp.float32),
            pltpu.VMEM((16 * NBINS,), jnp.float32),
            pltpu.VMEM((16 * NBINS,), jnp.float32),
            pltpu.VMEM((NBINS,), jnp.float32),
        ],
        compiler_params=pltpu.CompilerParams(needs_layout_passes=False),
    )
    return fn(bins)


def kernel(preds, true):
    true_i32 = true.astype(jnp.int32)
    bins = _tc_bins(preds, true_i32)
    rows = _sc_counts(bins)
    counts = rows.reshape(NW, NBINS).sum(axis=0).reshape(NCLS, NCLS)
    i = jnp.arange(NCLS, dtype=jnp.float32)
    weights = (i[:, None] - i[None, :]) ** 2 / float((NCLS - 1) ** 2)
    total = counts.sum()
    th = counts.sum(axis=1)
    ph = counts.sum(axis=0)
    num = (counts * weights).sum() / total
    e = jnp.outer(th, ph)
    den = (e * weights).sum() / e.sum()
    return num / den
